# Initial kernel scaffold; baseline (speedup 1.0000x reference)
import jax
import jax.numpy as jnp
from jax.experimental import pallas as pl


def kernel(x, edge_index, edge_attr, W_ne1, b_ne1, W_ne2, b_ne2, W_ee, b_ee, W_a1, b_a1, W_a2, b_a2):
    raise NotImplementedError("write your pallas kernel here")


# speedup vs baseline: 2.7287x; 2.7287x over previous
import functools

import jax
import jax.numpy as jnp
from jax import lax
from jax.experimental import pallas as pl
from jax.experimental.pallas import tpu as pltpu
from jax.experimental.pallas import tpu_sc as plsc

N = 100000
E = 1600000
D_NODE = 128
D_EDGE = 16
ENC = 8
HID = 16
OUT = 8

CH = 128
E2 = E // CH
NW = 32
BASE_ROWS = E2 // NW
EXTRA = E2 - BASE_ROWS * NW
STRIPE = N // 16


def _nodeenc_body(x_ref, w1_ref, b1_ref, w2_ref, b2_ref, o_ref):
    xb = x_ref[...]
    hmid = jnp.maximum(
        jnp.dot(xb, w1_ref[...], preferred_element_type=jnp.float32)
        + b1_ref[...], 0.0)
    o_ref[...] = (
        jnp.dot(hmid, w2_ref[...], preferred_element_type=jnp.float32)
        + b2_ref[...])


def _nodeenc(x, W1, b1, W2, b2):
    BN = 2000
    return pl.pallas_call(
        _nodeenc_body,
        grid=(N // BN,),
        in_specs=[
            pl.BlockSpec((BN, D_NODE), lambda i: (i, 0)),
            pl.BlockSpec((D_NODE, HID), lambda i: (0, 0)),
            pl.BlockSpec((1, HID), lambda i: (0, 0)),
            pl.BlockSpec((HID, ENC), lambda i: (0, 0)),
            pl.BlockSpec((1, ENC), lambda i: (0, 0)),
        ],
        out_specs=pl.BlockSpec((BN, ENC), lambda i: (i, 0)),
        out_shape=jax.ShapeDtypeStruct((N, ENC), jnp.float32),
    )(x, W1, b1, W2, b2)


def _msg_body(g_ref, ea_ref, w_ref, b_ref, o_ref):
    e = (jnp.dot(ea_ref[...], w_ref[...], preferred_element_type=jnp.float32)
         + b_ref[...])
    o_ref[...] = jnp.maximum(g_ref[...] + e, 0.0)


def _msg(g, edge_attr, W, b):
    BE = 12800
    return pl.pallas_call(
        _msg_body,
        grid=(E // BE,),
        in_specs=[
            pl.BlockSpec((BE, ENC), lambda i: (i, 0)),
            pl.BlockSpec((BE, D_EDGE), lambda i: (i, 0)),
            pl.BlockSpec((D_EDGE, ENC), lambda i: (0, 0)),
            pl.BlockSpec((1, ENC), lambda i: (0, 0)),
        ],
        out_specs=pl.BlockSpec((BE, ENC), lambda i: (i, 0)),
        out_shape=jax.ShapeDtypeStruct((E, ENC), jnp.float32),
    )(g, edge_attr, W, b)


def _final_body(h_ref, p_ref, w1_ref, b1_ref, w2_ref, b2_ref, o_ref):
    z = h_ref[...] + p_ref[0] + p_ref[1]
    zmid = jnp.maximum(
        jnp.dot(z, w1_ref[...], preferred_element_type=jnp.float32)
        + b1_ref[...], 0.0)
    o_ref[...] = (
        jnp.dot(zmid, w2_ref[...], preferred_element_type=jnp.float32)
        + b2_ref[...])


def _final(h, parts, W1, b1, W2, b2):
    BN = 2000
    return pl.pallas_call(
        _final_body,
        grid=(N // BN,),
        in_specs=[
            pl.BlockSpec((BN, ENC), lambda i: (i, 0)),
            pl.BlockSpec((2, BN, ENC), lambda i: (0, i, 0)),
            pl.BlockSpec((ENC, HID), lambda i: (0, 0)),
            pl.BlockSpec((1, HID), lambda i: (0, 0)),
            pl.BlockSpec((HID, OUT), lambda i: (0, 0)),
            pl.BlockSpec((1, OUT), lambda i: (0, 0)),
        ],
        out_specs=pl.BlockSpec((BN, OUT), lambda i: (i, 0)),
        out_shape=jax.ShapeDtypeStruct((N, OUT), jnp.float32),
    )(h, parts, W1, b1, W2, b2)


def _worker_range(c, s):
    wid = s * 2 + c
    r0 = wid * BASE_ROWS + jnp.minimum(wid, EXTRA)
    nr = jnp.where(wid < EXTRA, BASE_ROWS + 1, BASE_ROWS)
    return r0, nr


def _gather_sc(h, ei2):
    mesh = plsc.VectorSubcoreMesh(core_axis_name="c", subcore_axis_name="s")

    @functools.partial(
        pl.kernel, mesh=mesh,
        out_type=jax.ShapeDtypeStruct((E2, CH, ENC), jnp.float32),
        compiler_params=pltpu.CompilerParams(use_tc_tiling_on_sc=False),
        scratch_types=[
            pltpu.VMEM((CH,), jnp.int32),
            pltpu.VMEM((CH, ENC), jnp.float32),
            pltpu.SemaphoreType.DMA,
        ],
    )
    def k(h_hbm, ei_hbm, out_hbm, idx_v, rows_v, sem):
        c = lax.axis_index("c")
        s = lax.axis_index("s")
        r0, nr = _worker_range(c, s)

        def body(i, carry):
            r = r0 + i
            pltpu.sync_copy(ei_hbm.at[0, r], idx_v)
            pltpu.async_copy(h_hbm.at[idx_v], rows_v, sem).wait()
            pltpu.sync_copy(rows_v, out_hbm.at[r])
            return carry

        lax.fori_loop(0, nr, body, 0)

    return k(h, ei2)


def _scatter_sc(msg3, ei2, zeros):
    mesh = plsc.VectorSubcoreMesh(core_axis_name="c", subcore_axis_name="s")

    @functools.partial(
        pl.kernel, mesh=mesh,
        out_type=jax.ShapeDtypeStruct((2, N, ENC), jnp.float32),
        compiler_params=pltpu.CompilerParams(use_tc_tiling_on_sc=False),
        scratch_types=[
            pltpu.VMEM((CH,), jnp.int32),
            pltpu.VMEM((CH, ENC), jnp.float32),
            pltpu.VMEM_SHARED((N, ENC), jnp.float32),
        ],
    )
    def k(msg_hbm, ei_hbm, z_hbm, out_hbm, idx_v, rows_v, agg_sh):
        c = lax.axis_index("c")
        s = lax.axis_index("s")
        r0, nr = _worker_range(c, s)

        pltpu.sync_copy(z_hbm.at[pl.ds(s * STRIPE, STRIPE)],
                        agg_sh.at[pl.ds(s * STRIPE, STRIPE)])
        plsc.subcore_barrier()

        def body(i, carry):
            r = r0 + i
            pltpu.sync_copy(ei_hbm.at[1, r], idx_v)
            pltpu.sync_copy(msg_hbm.at[r], rows_v)
            pltpu.sync_copy(rows_v, agg_sh.at[idx_v], add=True)
            return carry

        lax.fori_loop(0, nr, body, 0)
        plsc.subcore_barrier()
        pltpu.sync_copy(agg_sh.at[pl.ds(s * STRIPE, STRIPE)],
                        out_hbm.at[c, pl.ds(s * STRIPE, STRIPE)])

    return k(msg3, ei2, zeros)


def kernel(x, edge_index, edge_attr,
           W_ne1, b_ne1, W_ne2, b_ne2,
           W_ee, b_ee,
           W_a1, b_a1, W_a2, b_a2):
    ei2 = edge_index.reshape(2, E2, CH)
    h = _nodeenc(x, W_ne1, b_ne1.reshape(1, HID), W_ne2, b_ne2.reshape(1, ENC))
    g = _gather_sc(h, ei2)
    msg = _msg(g.reshape(E, ENC), edge_attr, W_ee, b_ee.reshape(1, ENC))
    zeros = jnp.zeros((N, ENC), jnp.float32)
    parts = _scatter_sc(msg.reshape(E2, CH, ENC), ei2, zeros)
    return _final(h, parts, W_a1, b_a1.reshape(1, HID), W_a2, b_a2.reshape(1, OUT))


# speedup vs baseline: 5.2814x; 1.9355x over previous
import functools

import jax
import jax.numpy as jnp
from jax import lax
from jax.experimental import pallas as pl
from jax.experimental.pallas import tpu as pltpu
from jax.experimental.pallas import tpu_sc as plsc

N = 100000
E = 1600000
D_NODE = 128
D_EDGE = 16
ENC = 8
HID = 16
OUT = 8

CH = 128
E2 = E // CH
NW = 32
ROWS_PT = 390
TAIL0 = NW * ROWS_PT
K = 15
NSUP = ROWS_PT // K
STRIPE = N // 16


def _nodeenc_body(x_ref, w1_ref, b1_ref, w2_ref, b2_ref, o_ref):
    xb = x_ref[...]
    hmid = jnp.maximum(
        jnp.dot(xb, w1_ref[...], preferred_element_type=jnp.float32)
        + b1_ref[...], 0.0)
    o_ref[...] = (
        jnp.dot(hmid, w2_ref[...], preferred_element_type=jnp.float32)
        + b2_ref[...])


def _nodeenc(x, W1, b1, W2, b2):
    BN = 2000
    return pl.pallas_call(
        _nodeenc_body,
        grid=(N // BN,),
        in_specs=[
            pl.BlockSpec((BN, D_NODE), lambda i: (i, 0)),
            pl.BlockSpec((D_NODE, HID), lambda i: (0, 0)),
            pl.BlockSpec((1, HID), lambda i: (0, 0)),
            pl.BlockSpec((HID, ENC), lambda i: (0, 0)),
            pl.BlockSpec((1, ENC), lambda i: (0, 0)),
        ],
        out_specs=pl.BlockSpec((BN, ENC), lambda i: (i, 0)),
        out_shape=jax.ShapeDtypeStruct((N, ENC), jnp.float32),
    )(x, W1, b1, W2, b2)


def _edgeencT_body(eaT_ref, w_ref, b_ref, o_ref):
    et = lax.dot_general(w_ref[...], eaT_ref[...],
                         (((0,), (0,)), ((), ())),
                         preferred_element_type=jnp.float32)
    o_ref[...] = et + b_ref[...]


def _edgeencT(eaT, W, bcol):
    BE = 32000
    return pl.pallas_call(
        _edgeencT_body,
        grid=(E // BE,),
        in_specs=[
            pl.BlockSpec((D_EDGE, BE), lambda i: (0, i)),
            pl.BlockSpec((D_EDGE, ENC), lambda i: (0, 0)),
            pl.BlockSpec((ENC, 1), lambda i: (0, 0)),
        ],
        out_specs=pl.BlockSpec((ENC, BE), lambda i: (0, i)),
        out_shape=jax.ShapeDtypeStruct((ENC, E), jnp.float32),
    )(eaT, W, bcol)


def _final_body(p_ref, w1_ref, b1_ref, w2_ref, b2_ref, o_ref):
    z = p_ref[0] + p_ref[1]
    zmid = jnp.maximum(
        jnp.dot(z, w1_ref[...], preferred_element_type=jnp.float32)
        + b1_ref[...], 0.0)
    o_ref[...] = (
        jnp.dot(zmid, w2_ref[...], preferred_element_type=jnp.float32)
        + b2_ref[...])


def _final(parts128, W1big, b1big, W2big, b2big):
    NP = N * ENC // 128
    BZ = NP
    return pl.pallas_call(
        _final_body,
        grid=(NP // BZ,),
        in_specs=[
            pl.BlockSpec((2, BZ, 128), lambda i: (0, i, 0)),
            pl.BlockSpec((128, 256), lambda i: (0, 0)),
            pl.BlockSpec((1, 256), lambda i: (0, 0)),
            pl.BlockSpec((256, 128), lambda i: (0, 0)),
            pl.BlockSpec((1, 128), lambda i: (0, 0)),
        ],
        out_specs=pl.BlockSpec((BZ, 128), lambda i: (i, 0)),
        out_shape=jax.ShapeDtypeStruct((NP, 128), jnp.float32),
    )(parts128, W1big, b1big, W2big, b2big)


def _gine_sc(h, eT, ei2, zeros):
    mesh = plsc.VectorSubcoreMesh(core_axis_name="c", subcore_axis_name="s")

    @functools.partial(
        pl.kernel, mesh=mesh,
        out_type=jax.ShapeDtypeStruct((2, N, ENC), jnp.float32),
        compiler_params=pltpu.CompilerParams(use_tc_tiling_on_sc=False,
                                             needs_layout_passes=False),
        scratch_types=[
            pltpu.VMEM((K, CH), jnp.int32),
            pltpu.VMEM((K, CH), jnp.int32),
            pltpu.VMEM((ENC, K * CH), jnp.float32),
            pltpu.VMEM((K, CH, ENC), jnp.float32),
            pltpu.VMEM_SHARED((N, ENC), jnp.float32),
            pltpu.SemaphoreType.DMA,
            pltpu.SemaphoreType.DMA,
        ],
    )
    def k(h_hbm, eT_hbm, ei_hbm, z_hbm, out_hbm,
          sidx, didx, ebufT, rows, agg_sh, gsem, ssem):
        c = lax.axis_index("c")
        s = lax.axis_index("s")
        wid = s * 2 + c
        r0 = wid * ROWS_PT

        st = pl.ds(s * STRIPE, STRIPE)

        @pl.when(c == 0)
        def _():
            pltpu.sync_copy(h_hbm.at[st], agg_sh.at[st])

        @pl.when(c == 1)
        def _():
            pltpu.sync_copy(z_hbm.at[st], agg_sh.at[st])

        plsc.subcore_barrier()

        iot = lax.iota(jnp.int32, 16)
        pat = iot >> 3
        cidx = iot & 7

        def compute_row(rowref, colbase):
            def body(i, carry):
                ridx = pat + 2 * i
                hv = plsc.load_gather(rowref, [ridx, cidx])
                ev = plsc.load_gather(ebufT, [cidx, ridx + colbase])
                m = jnp.maximum(hv + ev, 0.0)
                plsc.store_scatter(rowref, [ridx, cidx], m)
                return carry
            lax.fori_loop(0, CH // 2, body, 0)

        def superchunk(t, carry):
            rr = r0 + t * K
            pltpu.sync_copy(ei_hbm.at[0, pl.ds(rr, K)], sidx)
            pltpu.sync_copy(ei_hbm.at[1, pl.ds(rr, K)], didx)
            pltpu.sync_copy(eT_hbm.at[:, pl.ds(rr * CH, K * CH)], ebufT)
            gds = [pltpu.async_copy(h_hbm.at[sidx.at[j]], rows.at[j], gsem)
                   for j in range(K)]
            for d in gds:
                d.wait()
            for j in range(K):
                compute_row(rows.at[j], j * CH)
            sds = [pltpu.async_copy(rows.at[j], agg_sh.at[didx.at[j]], ssem,
                                    add=True)
                   for j in range(K)]
            for d in sds:
                d.wait()
            return carry

        lax.fori_loop(0, NSUP, superchunk, 0)

        @pl.when(wid < E2 - TAIL0)
        def _():
            r = TAIL0 + wid
            pltpu.sync_copy(ei_hbm.at[0, r], sidx.at[0])
            pltpu.sync_copy(ei_hbm.at[1, r], didx.at[0])
            pltpu.sync_copy(eT_hbm.at[:, pl.ds(r * CH, CH)],
                            ebufT.at[:, pl.ds(0, CH)])
            pltpu.async_copy(h_hbm.at[sidx.at[0]], rows.at[0], gsem).wait()
            compute_row(rows.at[0], 0)
            pltpu.async_copy(rows.at[0], agg_sh.at[didx.at[0]], ssem,
                             add=True).wait()

        plsc.subcore_barrier()
        pltpu.sync_copy(agg_sh.at[st], out_hbm.at[c, st])

    return k(h, eT, ei2, zeros)


def kernel(x, edge_index, edge_attr,
           W_ne1, b_ne1, W_ne2, b_ne2,
           W_ee, b_ee,
           W_a1, b_a1, W_a2, b_a2):
    ei2 = edge_index.reshape(2, E2, CH)
    h = _nodeenc(x, W_ne1, b_ne1.reshape(1, HID), W_ne2, b_ne2.reshape(1, ENC))
    eT = _edgeencT(edge_attr.T, W_ee, b_ee.reshape(ENC, 1))
    zeros = jnp.zeros((N, ENC), jnp.float32)
    parts = _gine_sc(h, eT, ei2, zeros)
    eye = jnp.eye(16, dtype=jnp.float32)
    out128 = _final(parts.reshape(2, N * ENC // 128, 128),
                    jnp.kron(eye, W_a1), jnp.tile(b_a1, 16).reshape(1, 256),
                    jnp.kron(eye, W_a2), jnp.tile(b_a2, 16).reshape(1, 128))
    return out128.reshape(N, OUT)
